# baseline (speedup 1.0000x reference)
import jax
import jax.numpy as jnp
from jax.experimental import pallas as pl
from jax.experimental.pallas import tpu as pltpu

_T = 8
_G = 4


def _interlace_body(x_desc_ref, x_any, cwm_ref, wcw0_ref, wcw1_ref,
                    f1w_ref, f2w_ref, cb_ref, f1b_ref, f2b_ref, wcb_ref,
                    o_any, buf, sem_out, sem_pass):
    i = pl.program_id(0)
    nb = pl.num_programs(0)
    nf = x_desc_ref.shape[2]
    gc = nf // _G
    slot = i % 2

    pltpu.make_async_copy(x_any.at[i, :, nf:, :], o_any.at[i, :, nf:, :],
                          sem_pass.at[slot]).start()

    @pl.when(i >= 1)
    def _wait_prev_pass():
        pltpu.make_async_copy(x_any.at[i - 1, :, nf:, :],
                              o_any.at[i - 1, :, nf:, :],
                              sem_pass.at[(i - 1) % 2]).wait()

    @pl.when(i >= 2)
    def _wait_buf():
        pltpu.make_async_copy(buf.at[slot], o_any.at[i - 2, :, :nf, :],
                              sem_out.at[slot]).wait()

    data = x_desc_ref[0]
    pooled = jnp.mean(data, axis=2)

    def conv_t(mm, bias):
        a = mm[:, 0:1]
        b = mm[:, 1:2]
        c = mm[:, 2:3]
        z = jnp.zeros((1, 1), jnp.float32)
        return (b + jnp.concatenate([z, a[:-1]], axis=0)
                + jnp.concatenate([c[1:], z], axis=0) + bias)

    mm = jnp.dot(pooled, cwm_ref[...], preferred_element_type=jnp.float32)
    oc = conv_t(mm, cb_ref[0, 0])
    h1 = jnp.maximum(
        jnp.dot(f1w_ref[...], oc, preferred_element_type=jnp.float32)
        + f1b_ref[...], 0.0)
    o2 = (jnp.dot(f2w_ref[...], h1, preferred_element_type=jnp.float32)
          + f2b_ref[...])
    offv = 4.0 * (jax.nn.sigmoid(o2) - 0.5)

    wm0 = jnp.dot(pooled, wcw0_ref[...], preferred_element_type=jnp.float32)
    wm1 = jnp.dot(pooled, wcw1_ref[...], preferred_element_type=jnp.float32)
    xw0 = 2.0 * jax.nn.sigmoid(conv_t(wm0, wcb_ref[0, 0]))
    xw1 = 2.0 * jax.nn.sigmoid(conv_t(wm1, wcb_ref[1, 0]))

    for g in range(_G):
        off_g = offv[g % 2, 0]
        if g >= 2:
            off_g = -off_g
        o0f = jnp.floor(off_g)
        o0 = o0f.astype(jnp.int32)
        frac = off_g - o0f
        w0 = 1.0 - frac
        w1 = frac
        xw = xw0 if (g % 2 == 0) else xw1
        for t in range(_T):
            s0 = t + o0
            s1 = s0 + 1
            v0 = jnp.where((s0 >= 0) & (s0 < _T), 1.0, 0.0)
            v1 = jnp.where((s1 >= 0) & (s1 < _T), 1.0, 0.0)
            xwt = xw[t, 0]
            c0 = w0 * xwt * v0
            c1 = w1 * xwt * v1
            s0c = jnp.clip(s0, 0, _T - 1)
            s1c = jnp.clip(s1, 0, _T - 1)
            a0 = x_desc_ref[0, pl.ds(s0c, 1), g * gc:(g + 1) * gc, :]
            a1 = x_desc_ref[0, pl.ds(s1c, 1), g * gc:(g + 1) * gc, :]
            buf[slot, t, g * gc:(g + 1) * gc, :] = (c0 * a0 + c1 * a1)[0]

    pltpu.make_async_copy(buf.at[slot], o_any.at[i, :, :nf, :],
                          sem_out.at[slot]).start()

    @pl.when(i == nb - 1)
    def _drain():
        pltpu.make_async_copy(x_any.at[i, :, nf:, :], o_any.at[i, :, nf:, :],
                              sem_pass.at[slot]).wait()

        @pl.when(nb >= 2)
        def _drain_prev_out():
            pltpu.make_async_copy(buf.at[(i - 1) % 2],
                                  o_any.at[i - 1, :, :nf, :],
                                  sem_out.at[(i - 1) % 2]).wait()

        pltpu.make_async_copy(buf.at[slot], o_any.at[i, :, :nf, :],
                              sem_out.at[slot]).wait()


def kernel(x, off_conv_w, off_conv_b, off_fc1_w, off_fc1_b, off_fc2_w,
           off_fc2_b, w_conv_w, w_conv_b):
    n, c, h, w = x.shape
    nb = n // _T
    hw = h * w
    nf = c // 4
    xr = x.reshape(nb, _T, c, hw)

    cwm = off_conv_w[0]
    wcw0 = w_conv_w[0]
    wcw1 = w_conv_w[1]
    cb = off_conv_b.reshape(1, 1)
    f1b = off_fc1_b.reshape(_T, 1)
    f2b = off_fc2_b.reshape(2, 1)
    wcb = w_conv_b.reshape(2, 1)

    small = lambda shape: pl.BlockSpec(shape, lambda i: (0, 0))
    out = pl.pallas_call(
        _interlace_body,
        grid=(nb,),
        in_specs=[
            pl.BlockSpec((1, _T, nf, hw), lambda i: (i, 0, 0, 0)),
            pl.BlockSpec(memory_space=pl.ANY),
            small((nf, 3)), small((nf, 3)), small((nf, 3)),
            small((_T, _T)), small((2, _T)),
            small((1, 1)), small((_T, 1)), small((2, 1)), small((2, 1)),
        ],
        out_specs=pl.BlockSpec(memory_space=pl.ANY),
        out_shape=jax.ShapeDtypeStruct((nb, _T, c, hw), jnp.float32),
        scratch_shapes=[
            pltpu.VMEM((2, _T, nf, hw), jnp.float32),
            pltpu.SemaphoreType.DMA((2,)),
            pltpu.SemaphoreType.DMA((2,)),
        ],
        compiler_params=pltpu.CompilerParams(
            dimension_semantics=("arbitrary",)),
    )(xr, xr, cwm, wcw0, wcw1, off_fc1_w, off_fc2_w, cb, f1b, f2b, wcb)
    return out.reshape(n, c, h, w)


# speedup vs baseline: 2.5753x; 2.5753x over previous
import jax
import jax.numpy as jnp
from jax import lax
from jax.experimental import pallas as pl
from jax.experimental.pallas import tpu as pltpu
from jax.experimental.pallas import tpu_sc as plsc

_T = 8
_G = 4
_HW = 784
_C = 512
_NF = _C // 4
_NB = 8
_NELEM = _NB * _T * _C * _HW
_CH16 = 16 * _HW
_NW = 32
_NSUB = _NB * _T * _G * 2
_SUB_PW = _NSUB // _NW
_PASS_PW = (_NB * _T * (_C - _NF) // 16) // _NW
_PASS_PER_FRAME = (_C - _NF) // 16


def _param_body(x_ref, cwm_ref, wcw0_ref, wcw1_ref, f1w_ref, f2w_ref,
                cb_ref, f1b_ref, f2b_ref, wcb_ref,
                src0_ref, src1_ref, dst_ref, c0_ref, c1_ref):
    i = pl.program_id(0)
    data = x_ref[0]
    pooled = jnp.mean(data, axis=2)

    def conv_t(mm, bias):
        a = mm[:, 0:1]
        b = mm[:, 1:2]
        c = mm[:, 2:3]
        z = jnp.zeros((1, 1), jnp.float32)
        return (b + jnp.concatenate([z, a[:-1]], axis=0)
                + jnp.concatenate([c[1:], z], axis=0) + bias)

    mm = jnp.dot(pooled, cwm_ref[...], preferred_element_type=jnp.float32)
    oc = conv_t(mm, cb_ref[0, 0])
    h1 = jnp.maximum(
        jnp.dot(f1w_ref[...], oc, preferred_element_type=jnp.float32)
        + f1b_ref[...], 0.0)
    o2 = (jnp.dot(f2w_ref[...], h1, preferred_element_type=jnp.float32)
          + f2b_ref[...])
    offv = 4.0 * (jax.nn.sigmoid(o2) - 0.5)

    wm0 = jnp.dot(pooled, wcw0_ref[...], preferred_element_type=jnp.float32)
    wm1 = jnp.dot(pooled, wcw1_ref[...], preferred_element_type=jnp.float32)
    xw0 = 2.0 * jax.nn.sigmoid(conv_t(wm0, wcb_ref[0, 0]))
    xw1 = 2.0 * jax.nn.sigmoid(conv_t(wm1, wcb_ref[1, 0]))

    iota_t = lax.broadcasted_iota(jnp.int32, (_T, 1), 0)
    base_frame = i * _T

    for g in range(_G):
        off_g = offv[g % 2, 0]
        if g >= 2:
            off_g = -off_g
        o0f = jnp.floor(off_g)
        o0 = o0f.astype(jnp.int32)
        frac = off_g - o0f
        w0 = 1.0 - frac
        w1 = frac
        xw = xw0 if (g % 2 == 0) else xw1
        s0 = iota_t + o0
        s1 = s0 + 1
        v0 = jnp.where((s0 >= 0) & (s0 < _T), 1.0, 0.0)
        v1 = jnp.where((s1 >= 0) & (s1 < _T), 1.0, 0.0)
        s0c = jnp.clip(s0, 0, _T - 1)
        s1c = jnp.clip(s1, 0, _T - 1)
        c0col = w0 * xw * v0
        c1col = w1 * xw * v1
        for sub in range(2):
            gs = g * 2 + sub
            choff = g * 32 + sub * 16
            src0_ref[0, :, gs:gs + 1] = \
                ((base_frame + s0c) * _C + choff) * _HW
            src1_ref[0, :, gs:gs + 1] = \
                ((base_frame + s1c) * _C + choff) * _HW
            dst_ref[0, :, gs:gs + 1] = \
                ((base_frame + iota_t) * _C + choff) * _HW
            c0_ref[0, :, gs, :] = jnp.broadcast_to(c0col, (_T, 16))
            c1_ref[0, :, gs, :] = jnp.broadcast_to(c1col, (_T, 16))


def _tc_params(xr, cwm, wcw0, wcw1, f1w, f2w, cb, f1b, f2b, wcb):
    small = lambda shape: pl.BlockSpec(shape, lambda i: (0, 0))
    oblock = pl.BlockSpec((1, _T, _G * 2), lambda i: (i, 0, 0))
    oshape = jax.ShapeDtypeStruct((_NB, _T, _G * 2), jnp.int32)
    cblock = pl.BlockSpec((1, _T, _G * 2, 16), lambda i: (i, 0, 0, 0))
    cshape = jax.ShapeDtypeStruct((_NB, _T, _G * 2, 16), jnp.float32)
    return pl.pallas_call(
        _param_body,
        grid=(_NB,),
        in_specs=[
            pl.BlockSpec((1, _T, _NF, _HW), lambda i: (i, 0, 0, 0)),
            small((_NF, 3)), small((_NF, 3)), small((_NF, 3)),
            small((_T, _T)), small((2, _T)),
            small((1, 1)), small((_T, 1)), small((2, 1)), small((2, 1)),
        ],
        out_specs=[oblock, oblock, oblock, cblock, cblock],
        out_shape=[oshape, oshape, oshape, cshape, cshape],
        compiler_params=pltpu.CompilerParams(
            dimension_semantics=("arbitrary",)),
    )(xr, cwm, wcw0, wcw1, f1w, f2w, cb, f1b, f2b, wcb)


def _sc_body(x_hbm, src0_hbm, src1_hbm, dst_hbm, c0_hbm, c1_hbm, out_hbm,
             pbuf0, pbuf1, pbuf2, pbuf3, abuf0, abuf1, bbuf0, bbuf1,
             obuf0, obuf1, s0_v, s1_v, d_v, c0_v, c1_v,
             sem_pin, sem_pout, sem_a, sem_b, sem_o, sem_prm):
    pbufs = [pbuf0, pbuf1, pbuf2, pbuf3]
    abufs = [abuf0, abuf1]
    bbufs = [bbuf0, bbuf1]
    obufs = [obuf0, obuf1]
    wid = lax.axis_index("s") * 2 + lax.axis_index("c")

    oslice = pl.ds(wid * _SUB_PW, _SUB_PW)
    prm = [
        pltpu.make_async_copy(src0_hbm.at[oslice], s0_v, sem_prm),
        pltpu.make_async_copy(src1_hbm.at[oslice], s1_v, sem_prm),
        pltpu.make_async_copy(dst_hbm.at[oslice], d_v, sem_prm),
        pltpu.make_async_copy(c0_hbm.at[oslice], c0_v, sem_prm),
        pltpu.make_async_copy(c1_hbm.at[oslice], c1_v, sem_prm),
    ]
    for cp in prm:
        cp.start()

    qbase = wid * _PASS_PW

    def _poff(c):
        q = qbase + c
        f = q // _PASS_PER_FRAME
        s = q % _PASS_PER_FRAME
        return pl.multiple_of((f * _C + _NF + 16 * s) * _HW, 16)

    for c in range(4):
        pltpu.make_async_copy(x_hbm.at[pl.ds(_poff(c), _CH16)],
                              pbufs[c], sem_pin.at[c]).start()
    for c in range(_PASS_PW):
        p = c % 4
        off = _poff(c)
        pltpu.make_async_copy(x_hbm.at[pl.ds(off, _CH16)],
                              pbufs[p], sem_pin.at[p]).wait()
        out_cp = pltpu.make_async_copy(pbufs[p],
                                       out_hbm.at[pl.ds(off, _CH16)],
                                       sem_pout.at[p])
        out_cp.start()
        if c + 4 < _PASS_PW:
            out_cp.wait()
            pltpu.make_async_copy(x_hbm.at[pl.ds(_poff(c + 4), _CH16)],
                                  pbufs[p], sem_pin.at[p]).start()
    for c in range(_PASS_PW - 4, _PASS_PW):
        p = c % 4
        pltpu.make_async_copy(pbufs[p],
                              out_hbm.at[pl.ds(_poff(c), _CH16)],
                              sem_pout.at[p]).wait()

    for cp in prm:
        cp.wait()
    s0all = s0_v[...]
    s1all = s1_v[...]
    dall = d_v[...]

    def _in_copies(idx):
        p = idx % 2
        o0 = pl.multiple_of(s0all[idx], 16)
        o1 = pl.multiple_of(s1all[idx], 16)
        return (
            pltpu.make_async_copy(x_hbm.at[pl.ds(o0, _CH16)],
                                  abufs[p], sem_a.at[p]),
            pltpu.make_async_copy(x_hbm.at[pl.ds(o1, _CH16)],
                                  bbufs[p], sem_b.at[p]),
        )

    def _out_copy(idx):
        p = idx % 2
        od = pl.multiple_of(dall[idx], 16)
        return pltpu.make_async_copy(obufs[p],
                                     out_hbm.at[pl.ds(od, _CH16)],
                                     sem_o.at[p])

    for cp in _in_copies(0):
        cp.start()
    for idx in range(_SUB_PW):
        p = idx % 2
        if idx + 1 < _SUB_PW:
            for cp in _in_copies(idx + 1):
                cp.start()
        for cp in _in_copies(idx):
            cp.wait()
        if idx >= 2:
            _out_copy(idx - 2).wait()
        c0vec = c0_v[idx, :]
        c1vec = c1_v[idx, :]
        ab = abufs[p]
        bb = bbufs[p]
        ob = obufs[p]

        def _fma(cv, carry):
            sl = pl.ds(cv * 16, 16)
            ob[sl] = c0vec * ab[sl] + c1vec * bb[sl]
            return carry

        lax.fori_loop(0, _CH16 // 16, _fma, 0)
        _out_copy(idx).start()
    for idx in range(_SUB_PW - 2, _SUB_PW):
        _out_copy(idx).wait()


def _sc_datapath(x_flat, src0, src1, dst, c0, c1):
    mesh = plsc.VectorSubcoreMesh(core_axis_name="c", subcore_axis_name="s")
    f32 = jnp.float32
    i32 = jnp.int32
    return pl.kernel(
        _sc_body,
        out_type=jax.ShapeDtypeStruct((_NELEM,), f32),
        mesh=mesh,
        scratch_types=[
            pltpu.VMEM((_CH16,), f32),
            pltpu.VMEM((_CH16,), f32),
            pltpu.VMEM((_CH16,), f32),
            pltpu.VMEM((_CH16,), f32),
            pltpu.VMEM((_CH16,), f32),
            pltpu.VMEM((_CH16,), f32),
            pltpu.VMEM((_CH16,), f32),
            pltpu.VMEM((_CH16,), f32),
            pltpu.VMEM((_CH16,), f32),
            pltpu.VMEM((_CH16,), f32),
            pltpu.VMEM((_SUB_PW,), i32),
            pltpu.VMEM((_SUB_PW,), i32),
            pltpu.VMEM((_SUB_PW,), i32),
            pltpu.VMEM((_SUB_PW, 16), f32),
            pltpu.VMEM((_SUB_PW, 16), f32),
            pltpu.SemaphoreType.DMA((4,)),
            pltpu.SemaphoreType.DMA((4,)),
            pltpu.SemaphoreType.DMA((2,)),
            pltpu.SemaphoreType.DMA((2,)),
            pltpu.SemaphoreType.DMA((2,)),
            pltpu.SemaphoreType.DMA,
        ],
    )(x_flat, src0, src1, dst, c0, c1)


def kernel(x, off_conv_w, off_conv_b, off_fc1_w, off_fc1_b, off_fc2_w,
           off_fc2_b, w_conv_w, w_conv_b):
    n, c, h, w = x.shape
    xr = x.reshape(_NB, _T, c, h * w)

    cwm = off_conv_w[0]
    wcw0 = w_conv_w[0]
    wcw1 = w_conv_w[1]
    cb = off_conv_b.reshape(1, 1)
    f1b = off_fc1_b.reshape(_T, 1)
    f2b = off_fc2_b.reshape(2, 1)
    wcb = w_conv_b.reshape(2, 1)

    src0, src1, dst, c0, c1 = _tc_params(
        xr, cwm, wcw0, wcw1, off_fc1_w, off_fc2_w, cb, f1b, f2b, wcb)

    out = _sc_datapath(
        x.reshape(_NELEM),
        src0.reshape(_NSUB), src1.reshape(_NSUB), dst.reshape(_NSUB),
        c0.reshape(_NSUB, 16), c1.reshape(_NSUB, 16))
    return out.reshape(n, c, h, w)


# speedup vs baseline: 24.8567x; 9.6518x over previous
import jax
import jax.numpy as jnp
from jax import lax
from jax.experimental import pallas as pl
from jax.experimental.pallas import tpu as pltpu

_T = 8
_G = 4
_NB = 8
_F = 64
_C = 512
_NF = _C // 4
_HW = 784
_BHW = 16


def _pool_body(x_ref, o_ref):
    @pl.when(pl.program_id(0) == 0)
    def _init():
        o_ref[...] = jnp.zeros_like(o_ref)

    o_ref[...] += jnp.sum(x_ref[...], axis=0)


def _wmat_body(pooled_ref, cwm_ref, wcw0_ref, wcw1_ref, f1w_ref, f2w_ref,
               cb_ref, f1b_ref, f2b_ref, wcb_ref, w_ref):
    w_ref[...] = jnp.zeros_like(w_ref)

    def conv_t(mm, bias):
        a = mm[:, 0:1]
        b = mm[:, 1:2]
        c = mm[:, 2:3]
        z = jnp.zeros((1, 1), jnp.float32)
        return (b + jnp.concatenate([z, a[:-1]], axis=0)
                + jnp.concatenate([c[1:], z], axis=0) + bias)

    for n in range(_NB):
        pooled = pooled_ref[n * _T:(n + 1) * _T, :] * (1.0 / _HW)

        mm = jnp.dot(pooled, cwm_ref[...],
                     preferred_element_type=jnp.float32)
        oc = conv_t(mm, cb_ref[0, 0])
        h1 = jnp.maximum(
            jnp.dot(f1w_ref[...], oc, preferred_element_type=jnp.float32)
            + f1b_ref[...], 0.0)
        o2 = (jnp.dot(f2w_ref[...], h1, preferred_element_type=jnp.float32)
              + f2b_ref[...])
        offv = 4.0 * (jax.nn.sigmoid(o2) - 0.5)

        wm0 = jnp.dot(pooled, wcw0_ref[...],
                      preferred_element_type=jnp.float32)
        wm1 = jnp.dot(pooled, wcw1_ref[...],
                      preferred_element_type=jnp.float32)
        xw0 = 2.0 * jax.nn.sigmoid(conv_t(wm0, wcb_ref[0, 0]))
        xw1 = 2.0 * jax.nn.sigmoid(conv_t(wm1, wcb_ref[1, 0]))

        iota_t = lax.broadcasted_iota(jnp.int32, (_T, 1), 0)
        for g in range(_G):
            off_g = offv[g % 2, 0]
            if g >= 2:
                off_g = -off_g
            o0f = jnp.floor(off_g)
            o0 = o0f.astype(jnp.int32)
            frac = off_g - o0f
            w0 = 1.0 - frac
            w1 = frac
            xw = xw0 if (g % 2 == 0) else xw1
            s0 = iota_t + o0
            s1 = s0 + 1
            v0 = jnp.where((s0 >= 0) & (s0 < _T), 1.0, 0.0)
            v1 = jnp.where((s1 >= 0) & (s1 < _T), 1.0, 0.0)
            c0col = w0 * xw * v0
            c1col = w1 * xw * v1
            for d in range(-2, 3):
                m0 = jnp.where(o0 == d, 1.0, 0.0)
                m1 = jnp.where(o0 == d - 1, 1.0, 0.0)
                col = c0col * m0 + c1col * m1
                w_ref[d + 2, n * _T:(n + 1) * _T, g * 32:(g + 1) * 32] = (
                    jnp.broadcast_to(col, (_T, 32)))


def _data_body(x_ref, w_ref, o_ref):
    xb = x_ref[...]
    desc = xb[:, :, :_NF]
    acc = w_ref[2][None] * desc
    for d in (-2, -1, 1, 2):
        if d < 0:
            shifted = jnp.concatenate(
                [jnp.zeros((_BHW, -d, _NF), jnp.float32),
                 desc[:, :_F + d, :]], axis=1)
        else:
            shifted = jnp.concatenate(
                [desc[:, d:, :],
                 jnp.zeros((_BHW, d, _NF), jnp.float32)], axis=1)
        acc += w_ref[d + 2][None] * shifted
    o_ref[:, :, :_NF] = acc
    o_ref[:, :, _NF:] = xb[:, :, _NF:]


def kernel(x, off_conv_w, off_conv_b, off_fc1_w, off_fc1_b, off_fc2_w,
           off_fc2_b, w_conv_w, w_conv_b):
    n, c, h, w = x.shape
    hw = h * w
    xv = jnp.transpose(x, (2, 3, 0, 1)).reshape(hw, _F, _C)

    cwm = off_conv_w[0]
    wcw0 = w_conv_w[0]
    wcw1 = w_conv_w[1]
    cb = off_conv_b.reshape(1, 1)
    f1b = off_fc1_b.reshape(_T, 1)
    f2b = off_fc2_b.reshape(2, 1)
    wcb = w_conv_b.reshape(2, 1)

    pooled = pl.pallas_call(
        _pool_body,
        grid=(hw // _BHW,),
        in_specs=[pl.BlockSpec((_BHW, _F, _NF), lambda i: (i, 0, 0))],
        out_specs=pl.BlockSpec((_F, _NF), lambda i: (0, 0)),
        out_shape=jax.ShapeDtypeStruct((_F, _NF), jnp.float32),
        compiler_params=pltpu.CompilerParams(
            dimension_semantics=("arbitrary",)),
    )(xv)

    small = lambda shape: pl.BlockSpec(shape, lambda: (0, 0))
    wmat = pl.pallas_call(
        _wmat_body,
        in_specs=[
            small((_F, _NF)),
            small((_NF, 3)), small((_NF, 3)), small((_NF, 3)),
            small((_T, _T)), small((2, _T)),
            small((1, 1)), small((_T, 1)), small((2, 1)), small((2, 1)),
        ],
        out_specs=pl.BlockSpec((5, _F, _NF), lambda: (0, 0, 0)),
        out_shape=jax.ShapeDtypeStruct((5, _F, _NF), jnp.float32),
    )(pooled, cwm, wcw0, wcw1, off_fc1_w, off_fc2_w, cb, f1b, f2b, wcb)

    outv = pl.pallas_call(
        _data_body,
        grid=(hw // _BHW,),
        in_specs=[
            pl.BlockSpec((_BHW, _F, _C), lambda i: (i, 0, 0)),
            pl.BlockSpec((5, _F, _NF), lambda i: (0, 0, 0)),
        ],
        out_specs=pl.BlockSpec((_BHW, _F, _C), lambda i: (i, 0, 0)),
        out_shape=jax.ShapeDtypeStruct((hw, _F, _C), jnp.float32),
        compiler_params=pltpu.CompilerParams(
            dimension_semantics=("arbitrary",)),
    )(xv, wmat)

    return outv.reshape(h, w, n, c).transpose(2, 3, 0, 1)


# speedup vs baseline: 24.9029x; 1.0019x over previous
import jax
import jax.numpy as jnp
from jax import lax
from jax.experimental import pallas as pl
from jax.experimental.pallas import tpu as pltpu

_T = 8
_G = 4
_NB = 8
_F = 64
_C = 512
_NF = _C // 4
_HW = 784
_BHW = 16


def _wmat_body(x_ref, cwm_ref, wcw0_ref, wcw1_ref, f1w_ref, f2w_ref,
               cb_ref, f1b_ref, f2b_ref, wcb_ref, w_ref, acc_ref):
    i = pl.program_id(0)

    @pl.when(i == 0)
    def _init():
        acc_ref[...] = jnp.zeros_like(acc_ref)

    acc_ref[...] += jnp.sum(x_ref[...], axis=0)

    @pl.when(i == pl.num_programs(0) - 1)
    def _emit():
        _wmat_emit(acc_ref, cwm_ref, wcw0_ref, wcw1_ref, f1w_ref, f2w_ref,
                   cb_ref, f1b_ref, f2b_ref, wcb_ref, w_ref)


def _wmat_emit(pooled_ref, cwm_ref, wcw0_ref, wcw1_ref, f1w_ref, f2w_ref,
               cb_ref, f1b_ref, f2b_ref, wcb_ref, w_ref):
    w_ref[...] = jnp.zeros_like(w_ref)

    def conv_t(mm, bias):
        a = mm[:, 0:1]
        b = mm[:, 1:2]
        c = mm[:, 2:3]
        z = jnp.zeros((1, 1), jnp.float32)
        return (b + jnp.concatenate([z, a[:-1]], axis=0)
                + jnp.concatenate([c[1:], z], axis=0) + bias)

    for n in range(_NB):
        pooled = pooled_ref[n * _T:(n + 1) * _T, :] * (1.0 / _HW)

        mm = jnp.dot(pooled, cwm_ref[...],
                     preferred_element_type=jnp.float32)
        oc = conv_t(mm, cb_ref[0, 0])
        h1 = jnp.maximum(
            jnp.dot(f1w_ref[...], oc, preferred_element_type=jnp.float32)
            + f1b_ref[...], 0.0)
        o2 = (jnp.dot(f2w_ref[...], h1, preferred_element_type=jnp.float32)
              + f2b_ref[...])
        offv = 4.0 * (jax.nn.sigmoid(o2) - 0.5)

        wm0 = jnp.dot(pooled, wcw0_ref[...],
                      preferred_element_type=jnp.float32)
        wm1 = jnp.dot(pooled, wcw1_ref[...],
                      preferred_element_type=jnp.float32)
        xw0 = 2.0 * jax.nn.sigmoid(conv_t(wm0, wcb_ref[0, 0]))
        xw1 = 2.0 * jax.nn.sigmoid(conv_t(wm1, wcb_ref[1, 0]))

        iota_t = lax.broadcasted_iota(jnp.int32, (_T, 1), 0)
        for g in range(_G):
            off_g = offv[g % 2, 0]
            if g >= 2:
                off_g = -off_g
            o0f = jnp.floor(off_g)
            o0 = o0f.astype(jnp.int32)
            frac = off_g - o0f
            w0 = 1.0 - frac
            w1 = frac
            xw = xw0 if (g % 2 == 0) else xw1
            s0 = iota_t + o0
            s1 = s0 + 1
            v0 = jnp.where((s0 >= 0) & (s0 < _T), 1.0, 0.0)
            v1 = jnp.where((s1 >= 0) & (s1 < _T), 1.0, 0.0)
            c0col = w0 * xw * v0
            c1col = w1 * xw * v1
            for d in range(-2, 3):
                m0 = jnp.where(o0 == d, 1.0, 0.0)
                m1 = jnp.where(o0 == d - 1, 1.0, 0.0)
                col = c0col * m0 + c1col * m1
                w_ref[d + 2, n * _T:(n + 1) * _T, g * 32:(g + 1) * 32] = (
                    jnp.broadcast_to(col, (_T, 32)))


def _data_body(x_ref, w_ref, o_ref):
    xb = x_ref[...]
    desc = xb[:, :, :_NF]
    acc = w_ref[2][None] * desc
    for d in (-2, -1, 1, 2):
        if d < 0:
            shifted = jnp.concatenate(
                [jnp.zeros((_BHW, -d, _NF), jnp.float32),
                 desc[:, :_F + d, :]], axis=1)
        else:
            shifted = jnp.concatenate(
                [desc[:, d:, :],
                 jnp.zeros((_BHW, d, _NF), jnp.float32)], axis=1)
        acc += w_ref[d + 2][None] * shifted
    o_ref[:, :, :_NF] = acc
    o_ref[:, :, _NF:] = xb[:, :, _NF:]


def kernel(x, off_conv_w, off_conv_b, off_fc1_w, off_fc1_b, off_fc2_w,
           off_fc2_b, w_conv_w, w_conv_b):
    n, c, h, w = x.shape
    hw = h * w
    xv = jnp.transpose(x, (2, 3, 0, 1)).reshape(hw, _F, _C)

    cwm = off_conv_w[0]
    wcw0 = w_conv_w[0]
    wcw1 = w_conv_w[1]
    cb = off_conv_b.reshape(1, 1)
    f1b = off_fc1_b.reshape(_T, 1)
    f2b = off_fc2_b.reshape(2, 1)
    wcb = w_conv_b.reshape(2, 1)

    small = lambda shape: pl.BlockSpec(shape, lambda i: (0,) * len(shape))
    wmat = pl.pallas_call(
        _wmat_body,
        grid=(hw // _BHW,),
        in_specs=[
            pl.BlockSpec((_BHW, _F, _NF), lambda i: (i, 0, 0)),
            small((_NF, 3)), small((_NF, 3)), small((_NF, 3)),
            small((_T, _T)), small((2, _T)),
            small((1, 1)), small((_T, 1)), small((2, 1)), small((2, 1)),
        ],
        out_specs=pl.BlockSpec((5, _F, _NF), lambda i: (0, 0, 0)),
        out_shape=jax.ShapeDtypeStruct((5, _F, _NF), jnp.float32),
        scratch_shapes=[pltpu.VMEM((_F, _NF), jnp.float32)],
        compiler_params=pltpu.CompilerParams(
            dimension_semantics=("arbitrary",)),
    )(xv, cwm, wcw0, wcw1, off_fc1_w, off_fc2_w, cb, f1b, f2b, wcb)

    outv = pl.pallas_call(
        _data_body,
        grid=(hw // _BHW,),
        in_specs=[
            pl.BlockSpec((_BHW, _F, _C), lambda i: (i, 0, 0)),
            pl.BlockSpec((5, _F, _NF), lambda i: (0, 0, 0)),
        ],
        out_specs=pl.BlockSpec((_BHW, _F, _C), lambda i: (i, 0, 0)),
        out_shape=jax.ShapeDtypeStruct((hw, _F, _C), jnp.float32),
        compiler_params=pltpu.CompilerParams(
            dimension_semantics=("arbitrary",)),
    )(xv, wmat)

    return outv.reshape(h, w, n, c).transpose(2, 3, 0, 1)
